# Initial kernel scaffold; baseline (speedup 1.0000x reference)
import functools

import jax
import jax.numpy as jnp
from jax import lax
from jax.experimental import pallas as pl
from jax.experimental.pallas import tpu as pltpu
from jax.experimental.pallas import tpu_sc as plsc

N_NODES = 10000
N_EDGES = 320000
D = 128
N_CLASSES = 40
EPS = 0.3

NC = 2
NS = 16
L = 16
NW = NC * NS
NP = 10240
ROWS_T = NP // NS
E_W = N_EDGES // NW
CH = 80
NCHUNK = E_W // CH
ZR = 64

BR = 2048
GRID = NP // BR


def _sc_mesh():
    return plsc.VectorSubcoreMesh(
        core_axis_name="c", subcore_axis_name="s", num_cores=NC, num_subcores=NS
    )


def _deg_body(dst_hbm, deg_hbm, idx2, ones_v, zrow, deg_sp, sem):
    c = lax.axis_index("c")
    s = lax.axis_index("s")
    wid = s * NC + c

    @pl.loop(0, ROWS_T // L)
    def _zero_stamp(i):
        zrow[pl.ds(i * L, L)] = jnp.zeros((L,), jnp.float32)

    for i in range(CH // L):
        ones_v[pl.ds(i * L, L)] = jnp.ones((L,), jnp.float32)

    pltpu.sync_copy(zrow, deg_sp.at[pl.ds(s * ROWS_T, ROWS_T)])
    plsc.subcore_barrier()

    @pl.loop(0, NCHUNK)
    def _chunk(k):
        base = wid * E_W + k * CH
        pltpu.sync_copy(dst_hbm.at[pl.ds(base, CH)], idx2.at[0])
        pltpu.sync_copy(ones_v, deg_sp.at[idx2.at[0]], add=True)

    plsc.subcore_barrier()
    pltpu.sync_copy(
        deg_sp.at[pl.ds(s * ROWS_T, ROWS_T)],
        deg_hbm.at[c, pl.ds(s * ROWS_T, ROWS_T)],
    )


def _deg_call(dst):
    return pl.kernel(
        _deg_body,
        out_type=jax.ShapeDtypeStruct((NC, NP), jnp.float32),
        mesh=_sc_mesh(),
        scratch_types=[
            pltpu.VMEM((1, CH), jnp.int32),
            pltpu.VMEM((CH,), jnp.float32),
            pltpu.VMEM((ROWS_T,), jnp.float32),
            pltpu.VMEM_SHARED((NP,), jnp.float32),
            pltpu.SemaphoreType.DMA,
        ],
    )(dst)


def _layer_body(h_hbm, al_hbm, ar_hbm, dinv_hbm, src_hbm, dst_hbm, out_hbm,
                al_v, ar_v, dinv_v, idx_s, idx_d2, w_v, msg_v, zb, out_sp, sem):
    c = lax.axis_index("c")
    s = lax.axis_index("s")
    wid = s * NC + c

    pltpu.sync_copy(al_hbm, al_v)
    pltpu.sync_copy(ar_hbm, ar_v)
    pltpu.sync_copy(dinv_hbm, dinv_v)

    @pl.loop(0, ZR)
    def _zero_stamp(i):
        for j in range(D // L):
            zb[i, pl.ds(j * L, L)] = jnp.zeros((L,), jnp.float32)

    for r in range(ROWS_T // ZR):
        pltpu.sync_copy(zb, out_sp.at[pl.ds(s * ROWS_T + r * ZR, ZR)])
    plsc.subcore_barrier()

    @pl.loop(0, NCHUNK)
    def _chunk(k):
        base = wid * E_W + k * CH
        pltpu.sync_copy(src_hbm.at[pl.ds(base, CH)], idx_s)
        pltpu.sync_copy(dst_hbm.at[pl.ds(base, CH)], idx_d2.at[0])
        gather = pltpu.async_copy(h_hbm.at[idx_s], msg_v, sem)
        for g in range(CH // L):
            sl = pl.ds(g * L, L)
            si = idx_s[sl]
            di = idx_d2[0, sl]
            u = plsc.load_gather(al_v, [si]) + plsc.load_gather(ar_v, [di])
            a = jnp.abs(u)
            e = jnp.exp(-2.0 * a)
            t = jnp.sign(u) * (1.0 - e) / (1.0 + e)
            w_v[sl] = (t * plsc.load_gather(dinv_v, [si])
                       * plsc.load_gather(dinv_v, [di]))
        gather.wait()

        @pl.loop(0, CH)
        def _scale(ei):
            wv = lax.broadcast(w_v[ei], (L,))
            for j in range(D // L):
                sl = pl.ds(j * L, L)
                msg_v[ei, sl] = msg_v[ei, sl] * wv

        pltpu.sync_copy(msg_v, out_sp.at[idx_d2.at[0]], add=True)

    plsc.subcore_barrier()
    pltpu.sync_copy(
        out_sp.at[pl.ds(s * ROWS_T, ROWS_T)],
        out_hbm.at[c, pl.ds(s * ROWS_T, ROWS_T)],
    )


def _layer_call(h, al, ar, dinv, src, dst):
    return pl.kernel(
        _layer_body,
        out_type=jax.ShapeDtypeStruct((NC, NP, D), jnp.float32),
        mesh=_sc_mesh(),
        scratch_types=[
            pltpu.VMEM((NP,), jnp.float32),
            pltpu.VMEM((NP,), jnp.float32),
            pltpu.VMEM((NP,), jnp.float32),
            pltpu.VMEM((CH,), jnp.int32),
            pltpu.VMEM((1, CH), jnp.int32),
            pltpu.VMEM((CH,), jnp.float32),
            pltpu.VMEM((CH, D), jnp.float32),
            pltpu.VMEM((ZR, D), jnp.float32),
            pltpu.VMEM_SHARED((NP, D), jnp.float32),
            pltpu.SemaphoreType.DMA,
        ],
    )(h, al, ar, dinv, src, dst)


def _dense1_body(x_ref, w_ref, b_ref, attl_ref, attr_ref, degp_ref,
                 h_ref, al_ref, ar_ref, dinv_ref):
    z = lax.dot_general(x_ref[...], w_ref[...], (((1,), (1,)), ((), ())),
                        preferred_element_type=jnp.float32)
    h = jnp.maximum(z + b_ref[...][None, :], 0.0)
    h_ref[...] = h
    al_ref[...] = jnp.sum(h * attl_ref[...][None, :], axis=1, keepdims=True)
    ar_ref[...] = jnp.sum(h * attr_ref[...][None, :], axis=1, keepdims=True)
    deg = degp_ref[0] + degp_ref[1] + 1.0
    dinv_ref[...] = lax.rsqrt(deg)


def _dense1_call(x_p, t1_W, t1_b, attl, attr, degp3):
    return pl.pallas_call(
        _dense1_body,
        grid=(GRID,),
        in_specs=[
            pl.BlockSpec((BR, D), lambda i: (i, 0)),
            pl.BlockSpec((D, D), lambda i: (0, 0)),
            pl.BlockSpec((D,), lambda i: (0,)),
            pl.BlockSpec((D,), lambda i: (0,)),
            pl.BlockSpec((D,), lambda i: (0,)),
            pl.BlockSpec((2, BR, 1), lambda i: (0, i, 0)),
        ],
        out_specs=[
            pl.BlockSpec((BR, D), lambda i: (i, 0)),
            pl.BlockSpec((BR, 1), lambda i: (i, 0)),
            pl.BlockSpec((BR, 1), lambda i: (i, 0)),
            pl.BlockSpec((BR, 1), lambda i: (i, 0)),
        ],
        out_shape=[
            jax.ShapeDtypeStruct((NP, D), jnp.float32),
            jax.ShapeDtypeStruct((NP, 1), jnp.float32),
            jax.ShapeDtypeStruct((NP, 1), jnp.float32),
            jax.ShapeDtypeStruct((NP, 1), jnp.float32),
        ],
    )(x_p, t1_W, t1_b, attl, attr, degp3)


def _mid_body(outp_ref, h0_ref, al_ref, ar_ref, dinv_ref, attl_ref, attr_ref,
              h1_ref, al1_ref, ar1_ref):
    dv = dinv_ref[...]
    coef = jnp.tanh(al_ref[...] + ar_ref[...]) * dv * dv
    h0 = h0_ref[...]
    h1 = outp_ref[0] + outp_ref[1] + h0 * coef + EPS * h0
    h1_ref[...] = h1
    al1_ref[...] = jnp.sum(h1 * attl_ref[...][None, :], axis=1, keepdims=True)
    ar1_ref[...] = jnp.sum(h1 * attr_ref[...][None, :], axis=1, keepdims=True)


def _mid_call(outp, h0, al0, ar0, dinv, attl1, attr1):
    return pl.pallas_call(
        _mid_body,
        grid=(GRID,),
        in_specs=[
            pl.BlockSpec((2, BR, D), lambda i: (0, i, 0)),
            pl.BlockSpec((BR, D), lambda i: (i, 0)),
            pl.BlockSpec((BR, 1), lambda i: (i, 0)),
            pl.BlockSpec((BR, 1), lambda i: (i, 0)),
            pl.BlockSpec((BR, 1), lambda i: (i, 0)),
            pl.BlockSpec((D,), lambda i: (0,)),
            pl.BlockSpec((D,), lambda i: (0,)),
        ],
        out_specs=[
            pl.BlockSpec((BR, D), lambda i: (i, 0)),
            pl.BlockSpec((BR, 1), lambda i: (i, 0)),
            pl.BlockSpec((BR, 1), lambda i: (i, 0)),
        ],
        out_shape=[
            jax.ShapeDtypeStruct((NP, D), jnp.float32),
            jax.ShapeDtypeStruct((NP, 1), jnp.float32),
            jax.ShapeDtypeStruct((NP, 1), jnp.float32),
        ],
    )(outp, h0, al0, ar0, dinv, attl1, attr1)


def _fin_body(outp_ref, h0_ref, h1_ref, al_ref, ar_ref, dinv_ref,
              w2_ref, b2_ref, out_ref):
    dv = dinv_ref[...]
    coef = jnp.tanh(al_ref[...] + ar_ref[...]) * dv * dv
    h1 = h1_ref[...]
    h2 = outp_ref[0] + outp_ref[1] + h1 * coef + EPS * h0_ref[...]
    z = lax.dot_general(h2, w2_ref[...], (((1,), (1,)), ((), ())),
                        preferred_element_type=jnp.float32)
    z = z + b2_ref[...][None, :]
    m = jnp.max(z, axis=1, keepdims=True)
    zs = z - m
    out_ref[...] = zs - jnp.log(jnp.sum(jnp.exp(zs), axis=1, keepdims=True))


def _fin_call(outp, h0, h1, al1, ar1, dinv, t2_W, t2_b):
    return pl.pallas_call(
        _fin_body,
        grid=(GRID,),
        in_specs=[
            pl.BlockSpec((2, BR, D), lambda i: (0, i, 0)),
            pl.BlockSpec((BR, D), lambda i: (i, 0)),
            pl.BlockSpec((BR, D), lambda i: (i, 0)),
            pl.BlockSpec((BR, 1), lambda i: (i, 0)),
            pl.BlockSpec((BR, 1), lambda i: (i, 0)),
            pl.BlockSpec((BR, 1), lambda i: (i, 0)),
            pl.BlockSpec((N_CLASSES, D), lambda i: (0, 0)),
            pl.BlockSpec((N_CLASSES,), lambda i: (0,)),
        ],
        out_specs=pl.BlockSpec((BR, N_CLASSES), lambda i: (i, 0)),
        out_shape=jax.ShapeDtypeStruct((NP, N_CLASSES), jnp.float32),
    )(outp, h0, h1, al1, ar1, dinv, t2_W, t2_b)


def kernel(x, edge_index, t1_W, t1_b, att_l0, att_r0, att_l1, att_r1,
           t2_W, t2_b):
    src = edge_index[0]
    dst = edge_index[1]
    x_p = jnp.pad(x, ((0, NP - N_NODES), (0, 0)))

    degp = _deg_call(dst)
    h0, al0, ar0, dinv = _dense1_call(
        x_p, t1_W, t1_b, att_l0[:, 0], att_r0[:, 0], degp.reshape(NC, NP, 1))

    outp0 = _layer_call(h0, al0.reshape(NP), ar0.reshape(NP),
                        dinv.reshape(NP), src, dst)
    h1, al1, ar1 = _mid_call(outp0, h0, al0, ar0, dinv,
                             att_l1[:, 0], att_r1[:, 0])
    outp1 = _layer_call(h1, al1.reshape(NP), ar1.reshape(NP),
                        dinv.reshape(NP), src, dst)
    ls = _fin_call(outp1, h0, h1, al1, ar1, dinv, t2_W, t2_b)
    return ls[:N_NODES]


# speedup vs baseline: 18.9160x; 18.9160x over previous
import functools

import jax
import jax.numpy as jnp
from jax import lax
from jax.experimental import pallas as pl
from jax.experimental.pallas import tpu as pltpu
from jax.experimental.pallas import tpu_sc as plsc

N_NODES = 10000
N_EDGES = 320000
D = 128
N_CLASSES = 40
EPS = 0.3

NC = 2
NS = 16
L = 16
NW = NC * NS
NP = 10240
ROWS_T = NP // NS
E_W = N_EDGES // NW
CH = 80
NCHUNK = E_W // CH
ZR = 16

BR = 2048
GRID = NP // BR


def _sc_mesh():
    return plsc.VectorSubcoreMesh(
        core_axis_name="c", subcore_axis_name="s", num_cores=NC, num_subcores=NS
    )


def _deg_body(dst_hbm, deg_hbm, idx2, ones_v, zrow, deg_sp, sem):
    c = lax.axis_index("c")
    s = lax.axis_index("s")
    wid = s * NC + c

    @pl.loop(0, ROWS_T // L)
    def _zero_stamp(i):
        zrow[pl.ds(i * L, L)] = jnp.zeros((L,), jnp.float32)

    for i in range(CH // L):
        ones_v[pl.ds(i * L, L)] = jnp.ones((L,), jnp.float32)

    pltpu.sync_copy(zrow, deg_sp.at[pl.ds(s * ROWS_T, ROWS_T)])
    plsc.subcore_barrier()

    @pl.loop(0, NCHUNK)
    def _chunk(k):
        base = wid * E_W + k * CH
        pltpu.sync_copy(dst_hbm.at[pl.ds(base, CH)], idx2.at[0])
        pltpu.sync_copy(ones_v, deg_sp.at[idx2.at[0]], add=True)

    plsc.subcore_barrier()
    pltpu.sync_copy(
        deg_sp.at[pl.ds(s * ROWS_T, ROWS_T)],
        deg_hbm.at[c, pl.ds(s * ROWS_T, ROWS_T)],
    )


def _deg_call(dst):
    return pl.kernel(
        _deg_body,
        out_type=jax.ShapeDtypeStruct((NC, NP), jnp.float32),
        mesh=_sc_mesh(),
        scratch_types=[
            pltpu.VMEM((1, CH), jnp.int32),
            pltpu.VMEM((CH,), jnp.float32),
            pltpu.VMEM((ROWS_T,), jnp.float32),
            pltpu.VMEM_SHARED((NP,), jnp.float32),
            pltpu.SemaphoreType.DMA,
        ],
        compiler_params=pltpu.CompilerParams(needs_layout_passes=False),
    )(dst)


def _layer_body(h_hbm, al_hbm, ar_hbm, dinv_hbm, src_hbm, dst_hbm, out_hbm,
                al_v, ar_v, dinv_v, idx_s, idx_d2, w_v, msg_v, zb, out_sp, sem):
    c = lax.axis_index("c")
    s = lax.axis_index("s")
    wid = s * NC + c

    pltpu.sync_copy(al_hbm, al_v)
    pltpu.sync_copy(ar_hbm, ar_v)
    pltpu.sync_copy(dinv_hbm, dinv_v)

    @pl.loop(0, ZR)
    def _zero_stamp(i):
        for j in range(D // L):
            zb[i, pl.ds(j * L, L)] = jnp.zeros((L,), jnp.float32)

    for r in range(ROWS_T // ZR):
        pltpu.sync_copy(zb, out_sp.at[pl.ds(s * ROWS_T + r * ZR, ZR)])
    plsc.subcore_barrier()

    @pl.loop(0, NCHUNK)
    def _chunk(k):
        base = wid * E_W + k * CH
        pltpu.sync_copy(src_hbm.at[pl.ds(base, CH)], idx_s)
        pltpu.sync_copy(dst_hbm.at[pl.ds(base, CH)], idx_d2.at[0])
        gather = pltpu.async_copy(h_hbm.at[idx_s], msg_v, sem)
        for g in range(CH // L):
            sl = pl.ds(g * L, L)
            si = idx_s[sl]
            di = idx_d2[0, sl]
            u = plsc.load_gather(al_v, [si]) + plsc.load_gather(ar_v, [di])
            a = jnp.abs(u)
            e = jnp.exp(-2.0 * a)
            t = jnp.sign(u) * (1.0 - e) / (1.0 + e)
            w_v[sl] = (t * plsc.load_gather(dinv_v, [si])
                       * plsc.load_gather(dinv_v, [di]))
        gather.wait()

        @pl.loop(0, CH)
        def _scale(ei):
            ei_v = lax.broadcast(ei, (L,)).astype(jnp.int32)
            wv = plsc.load_gather(w_v, [ei_v])
            for j in range(D // L):
                sl = pl.ds(j * L, L)
                msg_v[ei, sl] = msg_v[ei, sl] * wv

        pltpu.sync_copy(msg_v, out_sp.at[idx_d2.at[0]], add=True)

    plsc.subcore_barrier()
    pltpu.sync_copy(
        out_sp.at[pl.ds(s * ROWS_T, ROWS_T)],
        out_hbm.at[c, pl.ds(s * ROWS_T, ROWS_T)],
    )


def _layer_call(h, al, ar, dinv, src, dst):
    return pl.kernel(
        _layer_body,
        out_type=jax.ShapeDtypeStruct((NC, NP, D), jnp.float32),
        mesh=_sc_mesh(),
        scratch_types=[
            pltpu.VMEM((NP,), jnp.float32),
            pltpu.VMEM((NP,), jnp.float32),
            pltpu.VMEM((NP,), jnp.float32),
            pltpu.VMEM((CH,), jnp.int32),
            pltpu.VMEM((1, CH), jnp.int32),
            pltpu.VMEM((CH,), jnp.float32),
            pltpu.VMEM((CH, D), jnp.float32),
            pltpu.VMEM((ZR, D), jnp.float32),
            pltpu.VMEM_SHARED((NP, D), jnp.float32),
            pltpu.SemaphoreType.DMA,
        ],
        compiler_params=pltpu.CompilerParams(needs_layout_passes=False),
    )(h, al, ar, dinv, src, dst)


def _dense1_body(x_ref, w_ref, b_ref, attl_ref, attr_ref, degp_ref,
                 h_ref, al_ref, ar_ref, dinv_ref):
    z = lax.dot_general(x_ref[...], w_ref[...], (((1,), (1,)), ((), ())),
                        preferred_element_type=jnp.float32)
    h = jnp.maximum(z + b_ref[...][None, :], 0.0)
    h_ref[...] = h
    al_ref[...] = jnp.sum(h * attl_ref[...][None, :], axis=1, keepdims=True)
    ar_ref[...] = jnp.sum(h * attr_ref[...][None, :], axis=1, keepdims=True)
    deg = degp_ref[0] + degp_ref[1] + 1.0
    dinv_ref[...] = lax.rsqrt(deg)


def _dense1_call(x_p, t1_W, t1_b, attl, attr, degp3):
    return pl.pallas_call(
        _dense1_body,
        grid=(GRID,),
        in_specs=[
            pl.BlockSpec((BR, D), lambda i: (i, 0)),
            pl.BlockSpec((D, D), lambda i: (0, 0)),
            pl.BlockSpec((D,), lambda i: (0,)),
            pl.BlockSpec((D,), lambda i: (0,)),
            pl.BlockSpec((D,), lambda i: (0,)),
            pl.BlockSpec((2, BR, 1), lambda i: (0, i, 0)),
        ],
        out_specs=[
            pl.BlockSpec((BR, D), lambda i: (i, 0)),
            pl.BlockSpec((BR, 1), lambda i: (i, 0)),
            pl.BlockSpec((BR, 1), lambda i: (i, 0)),
            pl.BlockSpec((BR, 1), lambda i: (i, 0)),
        ],
        out_shape=[
            jax.ShapeDtypeStruct((NP, D), jnp.float32),
            jax.ShapeDtypeStruct((NP, 1), jnp.float32),
            jax.ShapeDtypeStruct((NP, 1), jnp.float32),
            jax.ShapeDtypeStruct((NP, 1), jnp.float32),
        ],
    )(x_p, t1_W, t1_b, attl, attr, degp3)


def _mid_body(outp_ref, h0_ref, al_ref, ar_ref, dinv_ref, attl_ref, attr_ref,
              h1_ref, al1_ref, ar1_ref):
    dv = dinv_ref[...]
    coef = jnp.tanh(al_ref[...] + ar_ref[...]) * dv * dv
    h0 = h0_ref[...]
    h1 = outp_ref[0] + outp_ref[1] + h0 * coef + EPS * h0
    h1_ref[...] = h1
    al1_ref[...] = jnp.sum(h1 * attl_ref[...][None, :], axis=1, keepdims=True)
    ar1_ref[...] = jnp.sum(h1 * attr_ref[...][None, :], axis=1, keepdims=True)


def _mid_call(outp, h0, al0, ar0, dinv, attl1, attr1):
    return pl.pallas_call(
        _mid_body,
        grid=(GRID,),
        in_specs=[
            pl.BlockSpec((2, BR, D), lambda i: (0, i, 0)),
            pl.BlockSpec((BR, D), lambda i: (i, 0)),
            pl.BlockSpec((BR, 1), lambda i: (i, 0)),
            pl.BlockSpec((BR, 1), lambda i: (i, 0)),
            pl.BlockSpec((BR, 1), lambda i: (i, 0)),
            pl.BlockSpec((D,), lambda i: (0,)),
            pl.BlockSpec((D,), lambda i: (0,)),
        ],
        out_specs=[
            pl.BlockSpec((BR, D), lambda i: (i, 0)),
            pl.BlockSpec((BR, 1), lambda i: (i, 0)),
            pl.BlockSpec((BR, 1), lambda i: (i, 0)),
        ],
        out_shape=[
            jax.ShapeDtypeStruct((NP, D), jnp.float32),
            jax.ShapeDtypeStruct((NP, 1), jnp.float32),
            jax.ShapeDtypeStruct((NP, 1), jnp.float32),
        ],
    )(outp, h0, al0, ar0, dinv, attl1, attr1)


def _fin_body(outp_ref, h0_ref, h1_ref, al_ref, ar_ref, dinv_ref,
              w2_ref, b2_ref, out_ref):
    dv = dinv_ref[...]
    coef = jnp.tanh(al_ref[...] + ar_ref[...]) * dv * dv
    h1 = h1_ref[...]
    h2 = outp_ref[0] + outp_ref[1] + h1 * coef + EPS * h0_ref[...]
    z = lax.dot_general(h2, w2_ref[...], (((1,), (1,)), ((), ())),
                        preferred_element_type=jnp.float32)
    z = z + b2_ref[...][None, :]
    m = jnp.max(z, axis=1, keepdims=True)
    zs = z - m
    out_ref[...] = zs - jnp.log(jnp.sum(jnp.exp(zs), axis=1, keepdims=True))


def _fin_call(outp, h0, h1, al1, ar1, dinv, t2_W, t2_b):
    return pl.pallas_call(
        _fin_body,
        grid=(GRID,),
        in_specs=[
            pl.BlockSpec((2, BR, D), lambda i: (0, i, 0)),
            pl.BlockSpec((BR, D), lambda i: (i, 0)),
            pl.BlockSpec((BR, D), lambda i: (i, 0)),
            pl.BlockSpec((BR, 1), lambda i: (i, 0)),
            pl.BlockSpec((BR, 1), lambda i: (i, 0)),
            pl.BlockSpec((BR, 1), lambda i: (i, 0)),
            pl.BlockSpec((N_CLASSES, D), lambda i: (0, 0)),
            pl.BlockSpec((N_CLASSES,), lambda i: (0,)),
        ],
        out_specs=pl.BlockSpec((BR, N_CLASSES), lambda i: (i, 0)),
        out_shape=jax.ShapeDtypeStruct((NP, N_CLASSES), jnp.float32),
    )(outp, h0, h1, al1, ar1, dinv, t2_W, t2_b)


def kernel(x, edge_index, t1_W, t1_b, att_l0, att_r0, att_l1, att_r1,
           t2_W, t2_b):
    src = edge_index[0]
    dst = edge_index[1]
    x_p = jnp.pad(x, ((0, NP - N_NODES), (0, 0)))

    degp = _deg_call(dst)
    h0, al0, ar0, dinv = _dense1_call(
        x_p, t1_W, t1_b, att_l0[:, 0], att_r0[:, 0], degp.reshape(NC, NP, 1))

    outp0 = _layer_call(h0, al0.reshape(NP), ar0.reshape(NP),
                        dinv.reshape(NP), src, dst)
    h1, al1, ar1 = _mid_call(outp0, h0, al0, ar0, dinv,
                             att_l1[:, 0], att_r1[:, 0])
    outp1 = _layer_call(h1, al1.reshape(NP), ar1.reshape(NP),
                        dinv.reshape(NP), src, dst)
    ls = _fin_call(outp1, h0, h1, al1, ar1, dinv, t2_W, t2_b)
    return ls[:N_NODES]
